# Initial kernel scaffold; baseline (speedup 1.0000x reference)
import jax
import jax.numpy as jnp
from jax.experimental import pallas as pl


def kernel(r, e, a):
    raise NotImplementedError("write your pallas kernel here")


# speedup vs baseline: 6.1254x; 6.1254x over previous
import functools

import jax
import jax.numpy as jnp
from jax import lax
from jax.experimental import pallas as pl
from jax.experimental.pallas import tpu as pltpu
from jax.experimental.pallas import tpu_sc as plsc

N = 10000
E = 320000
D = 128

NC = 2
NS = 16
NW = NC * NS
EPW = E // NW
C = 80
NCHUNK = EPW // C
NLD = 10
RPS = N // NLD


def _sc_kernel_body(r_hbm, e_hbm, src_hbm, dst_hbm, z_hbm, out_hbm,
                    acc_sh, idx_s, idx_d, rows_s, rows_d, e_v, sem):
    c = lax.axis_index("c")
    s = lax.axis_index("s")
    wid = c * NS + s

    @pl.when(s < NLD)
    def _zero():
        pltpu.sync_copy(z_hbm.at[pl.ds(s * RPS, RPS)],
                        acc_sh.at[pl.ds(s * RPS, RPS)])

    plsc.subcore_barrier()

    base0 = wid * EPW

    def chunk_body(k, carry):
        base = base0 + k * C
        cp_s = pltpu.async_copy(src_hbm.at[pl.ds(base, C)], idx_s, sem)
        cp_d = pltpu.async_copy(dst_hbm.at[pl.ds(base, C)], idx_d, sem)
        cp_e = pltpu.async_copy(e_hbm.at[pl.ds(base, C)], e_v, sem)
        cp_s.wait()
        cp_d.wait()
        g_s = pltpu.async_copy(r_hbm.at[idx_s], rows_s, sem)
        g_d = pltpu.async_copy(r_hbm.at[idx_d], rows_d, sem)
        cp_e.wait()
        g_s.wait()
        g_d.wait()

        def mul_body(i, carry2):
            for j in range(D // 16):
                sl = pl.ds(j * 16, 16)
                ev = e_v[i, sl]
                rows_s[i, sl] = rows_s[i, sl] * ev
                rows_d[i, sl] = rows_d[i, sl] * ev
            return carry2

        lax.fori_loop(0, C, mul_body, 0, unroll=False)

        pltpu.sync_copy(rows_s, acc_sh.at[idx_d], add=True)
        pltpu.sync_copy(rows_d, acc_sh.at[idx_s], add=True)
        return carry

    lax.fori_loop(0, NCHUNK, chunk_body, 0, unroll=False)

    plsc.subcore_barrier()

    @pl.when(s < NLD)
    def _dump():
        pltpu.sync_copy(acc_sh.at[pl.ds(s * RPS, RPS)],
                        out_hbm.at[c, pl.ds(s * RPS, RPS)])


@jax.jit
def _message_passing_sc(r, e, src, dst, z):
    mesh = plsc.VectorSubcoreMesh(core_axis_name="c", subcore_axis_name="s")
    partials = pl.kernel(
        _sc_kernel_body,
        out_type=jax.ShapeDtypeStruct((NC, N, D), jnp.float32),
        mesh=mesh,
        scratch_types=[
            pltpu.VMEM_SHARED((N, D), jnp.float32),
            pltpu.VMEM((C,), jnp.int32),
            pltpu.VMEM((C,), jnp.int32),
            pltpu.VMEM((C, D), jnp.float32),
            pltpu.VMEM((C, D), jnp.float32),
            pltpu.VMEM((C, D), jnp.float32),
            pltpu.SemaphoreType.DMA,
        ],
    )(r, e, src, dst, z)
    return partials


def _add_body(a_ref, b_ref, o_ref):
    o_ref[...] = a_ref[...] + b_ref[...]


def _combine_tc(partials):
    return pl.pallas_call(
        _add_body,
        out_shape=jax.ShapeDtypeStruct((N, D), jnp.float32),
        grid=(10,),
        in_specs=[
            pl.BlockSpec((N // 10, D), lambda i: (i, 0)),
            pl.BlockSpec((N // 10, D), lambda i: (i, 0)),
        ],
        out_specs=pl.BlockSpec((N // 10, D), lambda i: (i, 0)),
    )(partials[0], partials[1])


def kernel(r, e, a):
    a = a.astype(jnp.int32)
    src = a[:, 0]
    dst = a[:, 1]
    z = jnp.zeros((N, D), jnp.float32)
    partials = _message_passing_sc(r, e, src, dst, z)
    return _combine_tc(partials)


# speedup vs baseline: 9.7877x; 1.5979x over previous
import jax
import jax.numpy as jnp
from jax import lax
from jax.experimental import pallas as pl
from jax.experimental.pallas import tpu as pltpu
from jax.experimental.pallas import tpu_sc as plsc

N = 10000
E = 320000
D = 128

NC = 2
NS = 16
NW = NC * NS
EPW = E // NW
C = 40
NCHUNK = EPW // C
NB = 3
NBI = 4
UNROLL = 12
NLD = 10
RPS = N // NLD


def _sc_kernel_body(r_hbm, e_hbm, src_hbm, dst_hbm, z_hbm, out_hbm,
                    acc_sh, idx_s, idx_d, e_v, g_s, g_d,
                    *sems):
    c = lax.axis_index("c")
    s = lax.axis_index("s")
    wid = c * NS + s
    sem_ld = sems[0:NB]
    sem_sc = sems[NB:2 * NB]
    sem_ix = sems[2 * NB:2 * NB + NBI]

    def idx_descs(k, bi):
        base = wid * EPW + k * C
        return (
            pltpu.make_async_copy(src_hbm.at[pl.ds(base, C)], idx_s.at[bi],
                                  sem_ix[bi]),
            pltpu.make_async_copy(dst_hbm.at[pl.ds(base, C)], idx_d.at[bi],
                                  sem_ix[bi]),
        )

    def load_descs(k, b, bi):
        base = wid * EPW + k * C
        return (
            pltpu.make_async_copy(e_hbm.at[pl.ds(base, C)], e_v.at[b],
                                  sem_ld[b]),
            pltpu.make_async_copy(r_hbm.at[idx_s.at[bi]], g_s.at[b],
                                  sem_ld[b]),
            pltpu.make_async_copy(r_hbm.at[idx_d.at[bi]], g_d.at[b],
                                  sem_ld[b]),
        )

    def scat_descs(k, b, bi):
        return (
            pltpu.make_async_copy(g_s.at[b], acc_sh.at[idx_d.at[bi]],
                                  sem_sc[b]),
            pltpu.make_async_copy(g_d.at[b], acc_sh.at[idx_s.at[bi]],
                                  sem_sc[b]),
        )

    for d in idx_descs(0, 0):
        d.start()
    for d in idx_descs(1, 1):
        d.start()
    for d in idx_descs(0, 0):
        d.wait()
    for d in load_descs(0, 0, 0):
        d.start()

    @pl.when(s < NLD)
    def _zero():
        pltpu.sync_copy(z_hbm.at[pl.ds(s * RPS, RPS)],
                        acc_sh.at[pl.ds(s * RPS, RPS)])

    plsc.subcore_barrier()

    def do_chunk(k, j, in_loop):
        b = j % NB
        bi = j % NBI

        def _drain():
            for d in scat_descs(k - 2, (b + 1) % NB, (bi + 2) % NBI):
                d.wait()

        if in_loop:
            pl.when(k >= 2)(_drain)
        else:
            _drain()

        if in_loop or k + 2 < NCHUNK:
            for d in idx_descs(k + 2, (bi + 2) % NBI):
                d.start()

        if in_loop or k + 1 < NCHUNK:
            for d in idx_descs(k + 1, (bi + 1) % NBI):
                d.wait()
            for d in load_descs(k + 1, (b + 1) % NB, (bi + 1) % NBI):
                d.start()

        for d in load_descs(k, b, bi):
            d.wait()

        ev_r = e_v.at[b]
        gs_r = g_s.at[b]
        gd_r = g_d.at[b]

        def mul_body(i, carry):
            for jj in range(D // 16):
                sl = pl.ds(jj * 16, 16)
                ev = ev_r[i, sl]
                gs_r[i, sl] = gs_r[i, sl] * ev
                gd_r[i, sl] = gd_r[i, sl] * ev
            return carry

        lax.fori_loop(0, C, mul_body, 0, unroll=False)

        pltpu.async_copy(g_s.at[b], acc_sh.at[idx_d.at[bi]], sem_sc[b],
                         add=True)
        pltpu.async_copy(g_d.at[b], acc_sh.at[idx_s.at[bi]], sem_sc[b],
                         add=True)

    NTAIL = NCHUNK % UNROLL
    NMAIN = NCHUNK - NTAIL

    @pl.loop(0, NMAIN, step=UNROLL)
    def _trips(k0):
        for j in range(UNROLL):
            do_chunk(k0 + j, j, True)

    for k in range(NMAIN, NCHUNK):
        do_chunk(k, k % UNROLL, False)

    k1, k2 = NCHUNK - 2, NCHUNK - 1
    for d in scat_descs(k1, k1 % NB, k1 % NBI):
        d.wait()
    for d in scat_descs(k2, k2 % NB, k2 % NBI):
        d.wait()

    plsc.subcore_barrier()

    @pl.when(s < NLD)
    def _dump():
        pltpu.sync_copy(acc_sh.at[pl.ds(s * RPS, RPS)],
                        out_hbm.at[c, pl.ds(s * RPS, RPS)])


@jax.jit
def _message_passing_sc(r, e, src, dst, z):
    mesh = plsc.VectorSubcoreMesh(core_axis_name="c", subcore_axis_name="s")
    partials = pl.kernel(
        _sc_kernel_body,
        out_type=jax.ShapeDtypeStruct((NC, N, D), jnp.float32),
        mesh=mesh,
        scratch_types=[
            pltpu.VMEM_SHARED((N, D), jnp.float32),
            pltpu.VMEM((NBI, C), jnp.int32),
            pltpu.VMEM((NBI, C), jnp.int32),
            pltpu.VMEM((NB, C, D), jnp.float32),
            pltpu.VMEM((NB, C, D), jnp.float32),
            pltpu.VMEM((NB, C, D), jnp.float32),
        ] + [pltpu.SemaphoreType.DMA] * (2 * NB + NBI),
    )(r, e, src, dst, z)
    return partials


def _add_body(a_ref, b_ref, o_ref):
    o_ref[...] = a_ref[...] + b_ref[...]


def _combine_tc(partials):
    return pl.pallas_call(
        _add_body,
        out_shape=jax.ShapeDtypeStruct((N, D), jnp.float32),
        grid=(10,),
        in_specs=[
            pl.BlockSpec((N // 10, D), lambda i: (i, 0)),
            pl.BlockSpec((N // 10, D), lambda i: (i, 0)),
        ],
        out_specs=pl.BlockSpec((N // 10, D), lambda i: (i, 0)),
    )(partials[0], partials[1])


def kernel(r, e, a):
    a = a.astype(jnp.int32)
    src = a[:, 0]
    dst = a[:, 1]
    z = jnp.zeros((N, D), jnp.float32)
    partials = _message_passing_sc(r, e, src, dst, z)
    return _combine_tc(partials)
